# baseline (speedup 1.0000x reference)
import functools
import math

import jax
import jax.numpy as jnp
from jax import lax
from jax.experimental import pallas as pl
from jax.experimental.pallas import tpu as pltpu
from jax.experimental.pallas import tpu_sc as plsc

N = 10000
E = 320000
D = 128

NC = 2
NS = 16
NW = NC * NS
L = 16

EPW = E // NW
C = 40
NCHUNK = EPW // C
RX = 2
RM = 3
RD = 6
ROWS_PER_TILE = 640
NPAD = NS * ROWS_PER_TILE
ZCOPIES = ROWS_PER_TILE // C

_SC_MESH = plsc.VectorSubcoreMesh(core_axis_name="c", subcore_axis_name="s")


@functools.partial(
    pl.kernel,
    out_type=jax.ShapeDtypeStruct((NC, NPAD, D), jnp.float32),
    mesh=_SC_MESH,
    scratch_types=[
        pltpu.VMEM((RX, C), jnp.int32),
        pltpu.VMEM((RD, C), jnp.int32),
        pltpu.VMEM((RX, C, D), jnp.float32),
        pltpu.VMEM((RM, C, D), jnp.float32),
        pltpu.VMEM_SHARED((NPAD, D), jnp.float32),
        pltpu.SemaphoreType.DMA((RX,)),
        pltpu.SemaphoreType.DMA((RM,)),
        pltpu.SemaphoreType.DMA((RM,)),
    ],
)
def _sc_aggregate(eidx_hbm, x_hbm, ea_hbm, out_hbm,
                  sidx, didx, xr, ms, agg, sem_p, sem_l, sem_s):
    cid = lax.axis_index("c")
    sid = lax.axis_index("s")
    wid = cid * NS + sid
    base_edge = wid * EPW
    row0 = sid * ROWS_PER_TILE

    zero = jnp.zeros((L,), jnp.float32)

    def zero_body(r, _):
        for j in range(D // L):
            ms[2, r, pl.ds(j * L, L)] = zero
        return 0

    lax.fori_loop(0, C, zero_body, 0)
    zcopies = [
        pltpu.async_copy(ms.at[2], agg.at[pl.ds(row0 + k * C, C)],
                         sem_s.at[2])
        for k in range(ZCOPIES)
    ]

    def issue_idx(i, bx, bd):
        pltpu.async_copy(eidx_hbm.at[0, wid, i], sidx.at[bx], sem_p.at[bx])
        pltpu.async_copy(eidx_hbm.at[1, wid, i], didx.at[bd], sem_p.at[bx])

    def wait_idx(i, bx, bd):
        pltpu.make_async_copy(eidx_hbm.at[0, wid, i], sidx.at[bx],
                              sem_p.at[bx]).wait()
        pltpu.make_async_copy(eidx_hbm.at[1, wid, i], didx.at[bd],
                              sem_p.at[bx]).wait()

    def issue_gather(bx, bm):
        pltpu.async_copy(x_hbm.at[sidx.at[bx]], xr.at[bx], sem_l.at[bm])

    def issue_ea(i, bm):
        eb = pl.multiple_of(base_edge + i * C, 8)
        pltpu.async_copy(ea_hbm.at[pl.ds(eb, C)], ms.at[bm], sem_l.at[bm])

    def issue_load(i, bx, bm):
        issue_gather(bx, bm)
        issue_ea(i, bm)

    def wait_load(i, bx, bm):
        eb = pl.multiple_of(base_edge + i * C, 8)
        pltpu.make_async_copy(x_hbm.at[sidx.at[bx]], xr.at[bx],
                              sem_l.at[bm]).wait()
        pltpu.make_async_copy(ea_hbm.at[pl.ds(eb, C)], ms.at[bm],
                              sem_l.at[bm]).wait()

    def issue_scatter(bm, bd):
        pltpu.async_copy(ms.at[bm], agg.at[didx.at[bd]], sem_s.at[bm],
                         add=True)

    def wait_scatter(bm, bd):
        pltpu.make_async_copy(ms.at[bm], agg.at[didx.at[bd]],
                              sem_s.at[bm]).wait()

    def relu(bx, bm):
        def relu_body(rr, _):
            for u in range(4):
                r = rr * 4 + u
                for k in range(D // L):
                    v = (xr[bx, r, pl.ds(k * L, L)]
                         + ms[bm, r, pl.ds(k * L, L)])
                    ms[bm, r, pl.ds(k * L, L)] = jnp.maximum(v, 0.0)
            return 0

        lax.fori_loop(0, C // 4, relu_body, 0)

    issue_idx(0, 0, 0)
    issue_idx(1, 1, 1)
    wait_idx(0, 0, 0)
    issue_load(0, 0, 0)
    for zc in zcopies:
        zc.wait()
    plsc.subcore_barrier()

    def chunk_step(i, x, xn, m, mn, d, dn, dn2, wait_sc, load_next,
                   next_idx):
        def guard(cond, fn):
            if isinstance(cond, bool):
                if cond:
                    fn()
            else:
                pl.when(cond)(fn)

        def _gather_head():
            wait_idx(i + 1, xn, dn)
            issue_gather(xn, mn)

        guard(load_next, _gather_head)
        guard(wait_sc, lambda: wait_scatter(mn, (d + RD - 2) % RD))
        guard(load_next, lambda: issue_ea(i + 1, mn))
        guard(next_idx, lambda: issue_idx(i + 2, x, dn2))
        wait_load(i, x, m)
        relu(x, m)
        issue_scatter(m, d)

    def pipeline_body(t, _):
        for j in range(6):
            i = t * 6 + j
            chunk_step(i, j % RX, (j + 1) % RX, j % RM, (j + 1) % RM,
                       j, (j + 1) % RD, (j + 2) % RD,
                       i >= 2 if j < 2 else True, True, True)
        return 0

    lax.fori_loop(0, NCHUNK // 6, pipeline_body, 0)

    i0 = (NCHUNK // 6) * 6
    chunk_step(i0, 0, 1, 0, 1, 0, 1, 2, True, True, True)
    chunk_step(i0 + 1, 1, 0, 1, 2, 1, 2, 3, True, True, True)
    chunk_step(i0 + 2, 0, 1, 2, 0, 2, 3, 4, True, True, False)
    chunk_step(i0 + 3, 1, 0, 0, 1, 3, 4, 5, True, False, False)

    wait_scatter((NCHUNK - 2) % RM, (NCHUNK - 2) % RD)
    wait_scatter((NCHUNK - 1) % RM, (NCHUNK - 1) % RD)
    plsc.subcore_barrier()
    pltpu.sync_copy(agg.at[pl.ds(row0, ROWS_PER_TILE)],
                    out_hbm.at[cid, pl.ds(row0, ROWS_PER_TILE)])


def _tc_body(x_ref, aggp_ref, g1_ref, be1_ref, w1t_ref, b1_ref,
             w2t_ref, b2_ref, g2_ref, be2_ref, out_ref):
    agg = aggp_ref[0, :N, :] + aggp_ref[1, :N, :]
    h = 2.0 * x_ref[...] + agg

    m1 = jnp.mean(h, axis=0, keepdims=True)
    d1 = h - m1
    v1 = jnp.mean(d1 * d1, axis=0, keepdims=True)
    f = d1 * lax.rsqrt(v1 + 1e-5) * g1_ref[...] + be1_ref[...]

    f = jnp.dot(f, w1t_ref[...], preferred_element_type=jnp.float32)
    f = f + b1_ref[...]
    f = 0.5 * f * (1.0 + lax.erf(f * (1.0 / math.sqrt(2.0))))
    f = jnp.dot(f, w2t_ref[...], preferred_element_type=jnp.float32)
    f = f + b2_ref[...]

    z = h + f
    m2 = jnp.mean(z, axis=0, keepdims=True)
    d2 = z - m2
    v2 = jnp.mean(d2 * d2, axis=0, keepdims=True)
    out_ref[...] = d2 * lax.rsqrt(v2 + 1e-5) * g2_ref[...] + be2_ref[...]


_tc_ffn = pl.pallas_call(
    _tc_body,
    out_shape=jax.ShapeDtypeStruct((N, D), jnp.float32),
)


@jax.jit
def kernel(x, edge_index, edge_attr, bn1_gamma, bn1_beta, W1, b1, W2, b2,
           bn2_gamma, bn2_beta):
    aggp = _sc_aggregate(edge_index.reshape(2, NW, NCHUNK, C), x, edge_attr)
    return _tc_ffn(x, aggp,
                   bn1_gamma.reshape(1, D), bn1_beta.reshape(1, D),
                   W1.T, b1.reshape(1, D),
                   W2.T, b2.reshape(1, D),
                   bn2_gamma.reshape(1, D), bn2_beta.reshape(1, D))
